# baseline (speedup 1.0000x reference)
import functools

import jax
import jax.numpy as jnp
from jax import lax
from jax.experimental import pallas as pl
from jax.experimental.pallas import tpu as pltpu
from jax.experimental.pallas import tpu_sc as plsc

VOCAB = 100000
LENGTH = 2048
FEATURES = 768

_NC = 2
_NS = 16
_NW = _NC * _NS
_LANES = 16
_B_PER_W = LENGTH // _NW
_CHUNK = _LANES
_NCHUNK = _B_PER_W // _CHUNK
_VECS_PER_ROW = FEATURES // _LANES


def _make_sc_kernel():
    mesh = plsc.VectorSubcoreMesh(core_axis_name="c", subcore_axis_name="s")

    @functools.partial(
        pl.kernel,
        mesh=mesh,
        out_type=jax.ShapeDtypeStruct((LENGTH, FEATURES), jnp.float32),
        scratch_types=[
            pltpu.VMEM((_B_PER_W,), jnp.int32),
            pltpu.VMEM((_B_PER_W, FEATURES), jnp.float32),
            pltpu.VMEM((_B_PER_W, FEATURES), jnp.float32),
        ]
        + [pltpu.SemaphoreType.DMA] * 3,
    )
    def emb_kernel(tokens_hbm, wte_hbm, wpe_hbm, out_hbm,
                   idx_v, rows_v, wpe_v, gsem, wsem, ssem):
        wid = lax.axis_index("s") * _NC + lax.axis_index("c")
        base = wid * _B_PER_W

        pltpu.sync_copy(tokens_hbm.at[pl.ds(base, _B_PER_W)], idx_v)

        half = _CHUNK // 2
        for p in range(2):
            row0 = p * half
            pltpu.async_copy(
                wte_hbm.at[idx_v.at[pl.ds(row0, half)]],
                rows_v.at[pl.ds(row0, half)], gsem)
            pltpu.async_copy(
                wpe_hbm.at[pl.ds(base + row0, half)],
                wpe_v.at[pl.ds(row0, half)], wsem)
        for k in range(1, _NCHUNK):
            row0 = k * _CHUNK
            idxs = idx_v[pl.ds(row0, _CHUNK)]
            pltpu.async_copy(
                wte_hbm.at[idxs], rows_v.at[pl.ds(row0, _CHUNK)], gsem)
            pltpu.async_copy(
                wpe_hbm.at[pl.ds(base + row0, _CHUNK)],
                wpe_v.at[pl.ds(row0, _CHUNK)], wsem)

        def add_row(r, c2):
            for c in range(_VECS_PER_ROW):
                sl = pl.ds(c * _LANES, _LANES)
                plsc.addupdate(rows_v.at[r, sl], wpe_v[r, sl])
            return c2

        def wait_rows(nrows):
            pltpu.make_async_copy(
                wte_hbm.at[pl.ds(0, nrows)],
                rows_v.at[pl.ds(0, nrows)], gsem).wait()
            pltpu.make_async_copy(
                wpe_hbm.at[pl.ds(0, nrows)],
                wpe_v.at[pl.ds(0, nrows)], wsem).wait()

        for p in range(2):
            row0 = p * half
            wait_rows(half)
            lax.fori_loop(row0, row0 + half, add_row, 0)
        pltpu.async_copy(
            rows_v.at[pl.ds(0, _CHUNK)],
            out_hbm.at[pl.ds(base, _CHUNK)], ssem)

        def chunk_body(k, carry):
            row0 = k * _CHUNK
            wait_rows(_CHUNK)
            lax.fori_loop(row0, row0 + _CHUNK, add_row, 0)
            pltpu.async_copy(
                rows_v.at[pl.ds(row0, _CHUNK)],
                out_hbm.at[pl.ds(base + row0, _CHUNK)], ssem)
            return carry

        lax.fori_loop(1, _NCHUNK, chunk_body, 0)
        for _ in range(_NCHUNK):
            pltpu.make_async_copy(
                rows_v.at[pl.ds(0, _CHUNK)],
                out_hbm.at[pl.ds(0, _CHUNK)], ssem).wait()

    return emb_kernel


_emb_kernel = _make_sc_kernel()


def kernel(tokens, wte, wpe):
    return _emb_kernel(tokens.astype(jnp.int32), wte, wpe)


# speedup vs baseline: 1.0472x; 1.0472x over previous
import functools

import jax
import jax.numpy as jnp
from jax import lax
from jax.experimental import pallas as pl
from jax.experimental.pallas import tpu as pltpu
from jax.experimental.pallas import tpu_sc as plsc

VOCAB = 100000
LENGTH = 2048
FEATURES = 768

_NC = 2
_NS = 16
_NW = _NC * _NS
_LANES = 16
_B_PER_W = LENGTH // _NW
_CHUNK = _LANES
_NCHUNK = _B_PER_W // _CHUNK
_VECS_PER_ROW = FEATURES // _LANES


def _make_sc_kernel():
    mesh = plsc.VectorSubcoreMesh(core_axis_name="c", subcore_axis_name="s")

    @functools.partial(
        pl.kernel,
        mesh=mesh,
        out_type=jax.ShapeDtypeStruct((LENGTH, FEATURES), jnp.float32),
        scratch_types=[
            pltpu.VMEM((_B_PER_W,), jnp.int32),
            pltpu.VMEM((_B_PER_W, FEATURES), jnp.float32),
            pltpu.VMEM((_B_PER_W, FEATURES), jnp.float32),
        ]
        + [pltpu.SemaphoreType.DMA] * 3,
    )
    def emb_kernel(tokens_hbm, wte_hbm, wpe_hbm, out_hbm,
                   idx_v, rows_v, wpe_v, gsem, wsem, ssem):
        wid = lax.axis_index("s") * _NC + lax.axis_index("c")
        base = wid * _B_PER_W

        pltpu.sync_copy(tokens_hbm.at[pl.ds(base, _B_PER_W)], idx_v)

        for k in range(_NCHUNK):
            row0 = k * _CHUNK
            idxs = idx_v[pl.ds(row0, _CHUNK)]
            pltpu.async_copy(
                wte_hbm.at[idxs], rows_v.at[pl.ds(row0, _CHUNK)], gsem)
            pltpu.async_copy(
                wpe_hbm.at[pl.ds(base + row0, _CHUNK)],
                wpe_v.at[pl.ds(row0, _CHUNK)], wsem)

        def chunk_body(k, carry):
            row0 = k * _CHUNK
            pltpu.make_async_copy(
                wte_hbm.at[pl.ds(0, _CHUNK)],
                rows_v.at[pl.ds(0, _CHUNK)], gsem).wait()
            pltpu.make_async_copy(
                wpe_hbm.at[pl.ds(0, _CHUNK)],
                wpe_v.at[pl.ds(0, _CHUNK)], wsem).wait()

            def add_row(r, c2):
                for c in range(_VECS_PER_ROW):
                    sl = pl.ds(c * _LANES, _LANES)
                    plsc.addupdate(rows_v.at[r, sl], wpe_v[r, sl])
                return c2

            lax.fori_loop(row0, row0 + _CHUNK, add_row, 0)
            pltpu.async_copy(
                rows_v.at[pl.ds(row0, _CHUNK)],
                out_hbm.at[pl.ds(base + row0, _CHUNK)], ssem)
            return carry

        lax.fori_loop(0, _NCHUNK, chunk_body, 0)
        for _ in range(_NCHUNK):
            pltpu.make_async_copy(
                rows_v.at[pl.ds(0, _CHUNK)],
                out_hbm.at[pl.ds(0, _CHUNK)], ssem).wait()

    return emb_kernel


_emb_kernel = _make_sc_kernel()


def kernel(tokens, wte, wpe):
    return _emb_kernel(tokens.astype(jnp.int32), wte, wpe)
